# Initial kernel scaffold; baseline (speedup 1.0000x reference)
import jax
import jax.numpy as jnp
from jax.experimental import pallas as pl


def kernel(inputs, dict_vals, jet_num):
    raise NotImplementedError("write your pallas kernel here")


# speedup vs baseline: 9.8099x; 9.8099x over previous
import functools
from itertools import combinations

import numpy as np
import jax
import jax.numpy as jnp
from jax import lax
from jax.experimental import pallas as pl
from jax.experimental.pallas import tpu as pltpu
from jax.experimental.pallas import tpu_sc as plsc

B, J, D = 1024, 16, 128
P = J * (J - 1) // 2
R = 4 * P
NC, NS = 2, 16
NW = NC * NS
BPW = B // NW
NCH = 5
CH = R // NCH
ZROW = J


def _build_lut() -> np.ndarray:
    lut = np.full((J - 1, 2 * P, 2), ZROW, np.int32)
    for n in range(2, J + 1):
        c = np.array(list(combinations(range(n), 2)), np.int32)
        m = c.shape[0]
        lut[n - 2, :m, :] = c
        lut[n - 2, m:2 * m, :] = c[:, ::-1]
    lut = lut.reshape(J - 1, R)
    return np.ascontiguousarray(
        np.pad(lut, ((0, 0), (0, RP - R)), constant_values=ZROW))


RP = 512
_LUT = _build_lut()

_MESH = plsc.VectorSubcoreMesh(core_axis_name="c", subcore_axis_name="s")


@functools.partial(
    pl.kernel,
    out_type=(
        jax.ShapeDtypeStruct((B * R, D), jnp.float32),
        jax.ShapeDtypeStruct((B,), jnp.float32),
    ),
    mesh=_MESH,
    scratch_types=[
        pltpu.VMEM((BPW,), jnp.int32),
        pltpu.VMEM((BPW,), jnp.float32),
        pltpu.VMEM((BPW // 16, 16), jnp.int32),
        pltpu.VMEM((16, RP), jnp.int32),
        [pltpu.VMEM((CH,), jnp.int32) for _ in range(NCH)],
        [pltpu.VMEM((CH, D), jnp.float32) for _ in range(NCH)],
        pltpu.SemaphoreType.DMA,
    ],
)
def _pairs_sc(table, lut, jn, out, out_pn,
              jn_v, pn_v, tidx_v, lutrows_v, idx_bufs, row_bufs, sem):
    wid = lax.axis_index("s") * NC + lax.axis_index("c")
    b0 = wid * BPW

    pltpu.sync_copy(jn.at[pl.ds(b0, BPW)], jn_v)

    for v in range(BPW // 16):
        jv = jn_v[pl.ds(v * 16, 16)]
        pn_v[pl.ds(v * 16, 16)] = (jv * (jv - 1)).astype(jnp.float32)
        tidx_v[v, :] = jv - 2
    pltpu.sync_copy(pn_v, out_pn.at[pl.ds(b0, BPW)])

    @pl.loop(0, BPW // 16)
    def _group(g):
        pltpu.async_copy(lut.at[tidx_v.at[g]], lutrows_v, sem).wait()
        for k in range(16):
            b = b0 + g * 16 + k
            boff = b * (J + 1)
            for v in range(R // 16):
                c, s = divmod(v, CH // 16)
                idx_bufs[c][pl.ds(s * 16, 16)] = (
                    lutrows_v[k, pl.ds(v * 16, 16)] + boff)
            cps = [pltpu.async_copy(table.at[idx_bufs[c]], row_bufs[c], sem)
                   for c in range(NCH)]
            for c in range(NCH):
                cps[c].wait()
                pltpu.sync_copy(row_bufs[c], out.at[pl.ds(b * R + c * CH, CH)])


def kernel(inputs, dict_vals, jet_num):
    del dict_vals
    jn32 = jet_num.astype(jnp.int32)
    table = jnp.concatenate(
        [inputs, jnp.zeros((B, 1, D), inputs.dtype)], axis=1
    ).reshape(B * (J + 1), D)
    out_rows, pn = _pairs_sc(table, jnp.asarray(_LUT), jn32)
    return out_rows.reshape(B, 2 * P, 2 * D), pn[:, None]


# speedup vs baseline: 19.4028x; 1.9779x over previous
import functools
from itertools import combinations

import numpy as np
import jax
import jax.numpy as jnp
from jax import lax
from jax.experimental import pallas as pl
from jax.experimental.pallas import tpu as pltpu
from jax.experimental.pallas import tpu_sc as plsc

B, J, D = 1024, 16, 128
P = J * (J - 1) // 2
R = 4 * P
RP = 512
NC, NS = 2, 16
NW = NC * NS
BPW = B // NW
NCH = 5
CH = R // NCH
NB = 6
PD = 2
ZROW = J
NJOB = BPW * NCH


def _build_lut() -> np.ndarray:
    lut = np.full((J - 1, 2 * P, 2), ZROW, np.int32)
    for n in range(2, J + 1):
        c = np.array(list(combinations(range(n), 2)), np.int32)
        m = c.shape[0]
        lut[n - 2, :m, :] = c
        lut[n - 2, m:2 * m, :] = c[:, ::-1]
    lut = lut.reshape(J - 1, R)
    lut = np.pad(lut, ((0, 0), (0, RP - R)), constant_values=ZROW)
    return np.ascontiguousarray(lut.reshape((J - 1) * (RP // D), D))


_LUT = _build_lut()

_MESH = plsc.VectorSubcoreMesh(core_axis_name="c", subcore_axis_name="s")


@functools.partial(
    pl.kernel,
    out_type=(
        jax.ShapeDtypeStruct((B * R, D), jnp.float32),
        jax.ShapeDtypeStruct((B,), jnp.float32),
    ),
    mesh=_MESH,
    scratch_types=[
        pltpu.VMEM((BPW,), jnp.int32),
        pltpu.VMEM((BPW,), jnp.float32),
        pltpu.VMEM((4 * 16,), jnp.int32),
        pltpu.VMEM((4 * 16, D), jnp.int32),
        [pltpu.VMEM((CH,), jnp.int32) for _ in range(NB)],
        [pltpu.VMEM((CH, D), jnp.float32) for _ in range(NB)],
        pltpu.VMEM((CH,), jnp.int32),
        pltpu.VMEM((CH, D), jnp.float32),
        pltpu.SemaphoreType.DMA,
        [pltpu.SemaphoreType.DMA for _ in range(NB)],
        [pltpu.SemaphoreType.DMA for _ in range(NB)],
    ],
)
def _pairs_sc(table, lut, jn, out, out_pn,
              jn_v, pn_v, tidx_v, lutrows_v, idx_bufs, row_bufs,
              zidx_v, zbuf, lsem, gsems, wsems):
    wid = lax.axis_index("s") * NC + lax.axis_index("c")
    b0 = wid * BPW

    pltpu.sync_copy(jn.at[pl.ds(b0, BPW)], jn_v)

    nn_vecs = []
    for v in range(BPW // 16):
        jv = jn_v[pl.ds(v * 16, 16)]
        pn_v[pl.ds(v * 16, 16)] = (jv * (jv - 1)).astype(jnp.float32)
        nn_vecs.append(jv * (jv - 1))
    pltpu.sync_copy(pn_v, out_pn.at[pl.ds(b0, BPW)])

    zrow = b0 * (J + 1) + ZROW
    for u in range(CH // 16):
        zidx_v[pl.ds(u * 16, 16)] = jnp.zeros((16,), jnp.int32) + zrow
    pltpu.async_copy(table.at[zidx_v], zbuf, lsem).wait()

    nn_s = [None] * BPW
    gdesc = [None] * NJOB
    wdesc = [None] * NJOB

    def fetch_lut_group(g):
        jv = jn_v[pl.ds(g * 16, 16)]
        for i in range(4):
            tidx_v[pl.ds(i * 16, 16)] = 4 * (jv - 2) + i
        pltpu.async_copy(lut.at[tidx_v], lutrows_v, lsem).wait()

    def complete(j):
        ev, c = divmod(j, NCH)
        s = j % NB
        b = b0 + ev
        dst = out.at[pl.ds(b * R + c * CH, CH)]
        if c == 0:
            gdesc[j].wait()
            wdesc[j] = pltpu.async_copy(row_bufs[s], dst, wsems[s])
        else:
            nn = nn_s[ev]

            @pl.when(nn > 48 * c)
            def _():
                gdesc[j].wait()
                pltpu.async_copy(row_bufs[s], dst, wsems[s])

            @pl.when(nn <= 48 * c)
            def _():
                pltpu.async_copy(zbuf, dst, wsems[s])

            wdesc[j] = pltpu.make_async_copy(zbuf, dst, wsems[s])

    for j in range(NJOB):
        ev, c = divmod(j, NCH)
        s = j % NB
        g, k = divmod(ev, 16)
        if c == 0:
            if k == 0:
                fetch_lut_group(g)
            nn_s[ev] = nn_vecs[g][k]
        if j >= NB:
            wdesc[j - NB].wait()

        boff = (b0 + ev) * (J + 1)

        def build_and_gather(c=c, s=s, k=k, boff=boff):
            for u in range(CH // 16):
                v = c * (CH // 16) + u
                i, w = divmod(v, D // 16)
                idx_bufs[s][pl.ds(u * 16, 16)] = (
                    lutrows_v[i * 16 + k, pl.ds(w * 16, 16)] + boff)

        dsc = pltpu.make_async_copy(table.at[idx_bufs[s]], row_bufs[s],
                                    gsems[s])
        if c == 0:
            build_and_gather()
            dsc.start()
        else:
            @pl.when(nn_s[ev] > 48 * c)
            def _(dsc=dsc):
                build_and_gather()
                dsc.start()
        gdesc[j] = dsc

        if j >= PD:
            complete(j - PD)

    for j in range(NJOB - PD, NJOB):
        complete(j)
    for j in range(NJOB - NB, NJOB):
        wdesc[j].wait()


def kernel(inputs, dict_vals, jet_num):
    del dict_vals
    jn32 = jet_num.astype(jnp.int32)
    table = jnp.concatenate(
        [inputs, jnp.zeros((B, 1, D), inputs.dtype)], axis=1
    ).reshape(B * (J + 1), D)
    out_rows, pn = _pairs_sc(table, jnp.asarray(_LUT), jn32)
    return out_rows.reshape(B, 2 * P, 2 * D), pn[:, None]


# speedup vs baseline: 20.2261x; 1.0424x over previous
import functools
from itertools import combinations

import numpy as np
import jax
import jax.numpy as jnp
from jax import lax
from jax.experimental import pallas as pl
from jax.experimental.pallas import tpu as pltpu
from jax.experimental.pallas import tpu_sc as plsc

B, J, D = 1024, 16, 128
P = J * (J - 1) // 2
R = 4 * P
RP = 512
NC, NS = 2, 16
NW = NC * NS
BPW = B // NW
NCH = 5
CH = R // NCH
NB = 8
PD = 4
ZROW = J
NJOB = BPW * NCH


def _build_lut() -> np.ndarray:
    lut = np.full((J - 1, 2 * P, 2), ZROW, np.int32)
    for n in range(2, J + 1):
        c = np.array(list(combinations(range(n), 2)), np.int32)
        m = c.shape[0]
        lut[n - 2, :m, :] = c
        lut[n - 2, m:2 * m, :] = c[:, ::-1]
    lut = lut.reshape(J - 1, R)
    lut = np.pad(lut, ((0, 0), (0, RP - R)), constant_values=ZROW)
    return np.ascontiguousarray(lut.reshape((J - 1) * (RP // D), D))


_LUT = _build_lut()

_MESH = plsc.VectorSubcoreMesh(core_axis_name="c", subcore_axis_name="s")


@functools.partial(
    pl.kernel,
    out_type=(
        jax.ShapeDtypeStruct((B * R, D), jnp.float32),
        jax.ShapeDtypeStruct((B,), jnp.float32),
    ),
    mesh=_MESH,
    scratch_types=[
        pltpu.VMEM((BPW,), jnp.int32),
        pltpu.VMEM((BPW,), jnp.float32),
        pltpu.VMEM((4 * 16,), jnp.int32),
        pltpu.VMEM((4 * 16, D), jnp.int32),
        [pltpu.VMEM((CH,), jnp.int32) for _ in range(NB)],
        [pltpu.VMEM((CH, D), jnp.float32) for _ in range(NB)],
        pltpu.VMEM((CH,), jnp.int32),
        pltpu.VMEM((CH, D), jnp.float32),
        pltpu.SemaphoreType.DMA,
        [pltpu.SemaphoreType.DMA for _ in range(NB)],
        [pltpu.SemaphoreType.DMA for _ in range(NB)],
    ],
)
def _pairs_sc(table, lut, jn, out, out_pn,
              jn_v, pn_v, tidx_v, lutrows_v, idx_bufs, row_bufs,
              zidx_v, zbuf, lsem, gsems, wsems):
    wid = lax.axis_index("s") * NC + lax.axis_index("c")
    b0 = wid * BPW

    pltpu.sync_copy(jn.at[pl.ds(b0, BPW)], jn_v)

    nn_vecs = []
    for v in range(BPW // 16):
        jv = jn_v[pl.ds(v * 16, 16)]
        pn_v[pl.ds(v * 16, 16)] = (jv * (jv - 1)).astype(jnp.float32)
        nn_vecs.append(jv * (jv - 1))
    pltpu.sync_copy(pn_v, out_pn.at[pl.ds(b0, BPW)])

    zrow = b0 * (J + 1) + ZROW
    for u in range(CH // 16):
        zidx_v[pl.ds(u * 16, 16)] = jnp.zeros((16,), jnp.int32) + zrow
    pltpu.async_copy(table.at[zidx_v], zbuf, lsem).wait()

    nn_s = [None] * BPW
    gdesc = [None] * NJOB
    wdesc = [None] * NJOB

    def fetch_lut_group(g):
        jv = jn_v[pl.ds(g * 16, 16)]
        for i in range(4):
            tidx_v[pl.ds(i * 16, 16)] = 4 * (jv - 2) + i
        pltpu.async_copy(lut.at[tidx_v], lutrows_v, lsem).wait()

    def complete(j):
        ev, c = divmod(j, NCH)
        s = j % NB
        b = b0 + ev
        dst = out.at[pl.ds(b * R + c * CH, CH)]
        if c == 0:
            gdesc[j].wait()
            wdesc[j] = pltpu.async_copy(row_bufs[s], dst, wsems[s])
        else:
            nn = nn_s[ev]

            @pl.when(nn > 48 * c)
            def _():
                gdesc[j].wait()
                pltpu.async_copy(row_bufs[s], dst, wsems[s])

            @pl.when(nn <= 48 * c)
            def _():
                pltpu.async_copy(zbuf, dst, wsems[s])

            wdesc[j] = pltpu.make_async_copy(zbuf, dst, wsems[s])

    for j in range(NJOB):
        ev, c = divmod(j, NCH)
        s = j % NB
        g, k = divmod(ev, 16)
        if c == 0:
            if k == 0:
                fetch_lut_group(g)
            nn_s[ev] = nn_vecs[g][k]
        if j >= NB:
            wdesc[j - NB].wait()

        boff = (b0 + ev) * (J + 1)

        def build_and_gather(c=c, s=s, k=k, boff=boff):
            for u in range(CH // 16):
                v = c * (CH // 16) + u
                i, w = divmod(v, D // 16)
                idx_bufs[s][pl.ds(u * 16, 16)] = (
                    lutrows_v[i * 16 + k, pl.ds(w * 16, 16)] + boff)

        dsc = pltpu.make_async_copy(table.at[idx_bufs[s]], row_bufs[s],
                                    gsems[s])
        if c == 0:
            build_and_gather()
            dsc.start()
        else:
            @pl.when(nn_s[ev] > 48 * c)
            def _(dsc=dsc):
                build_and_gather()
                dsc.start()
        gdesc[j] = dsc

        if j >= PD:
            complete(j - PD)

    for j in range(NJOB - PD, NJOB):
        complete(j)
    for j in range(NJOB - NB, NJOB):
        wdesc[j].wait()


def kernel(inputs, dict_vals, jet_num):
    del dict_vals
    jn32 = jet_num.astype(jnp.int32)
    table = jnp.concatenate(
        [inputs, jnp.zeros((B, 1, D), inputs.dtype)], axis=1
    ).reshape(B * (J + 1), D)
    out_rows, pn = _pairs_sc(table, jnp.asarray(_LUT), jn32)
    return out_rows.reshape(B, 2 * P, 2 * D), pn[:, None]


# speedup vs baseline: 21.3526x; 1.0557x over previous
import functools
from itertools import combinations

import numpy as np
import jax
import jax.numpy as jnp
from jax import lax
from jax.experimental import pallas as pl
from jax.experimental.pallas import tpu as pltpu
from jax.experimental.pallas import tpu_sc as plsc

B, J, D = 1024, 16, 128
P = J * (J - 1) // 2
R = 4 * P
RP = 512
NC, NS = 2, 16
NW = NC * NS
BPW = B // NW
NCH = 5
CH = R // NCH
NB = 8
PD = 6
ZROW = J
NJOB = BPW * NCH


def _build_lut() -> np.ndarray:
    lut = np.full((J - 1, 2 * P, 2), ZROW, np.int32)
    for n in range(2, J + 1):
        c = np.array(list(combinations(range(n), 2)), np.int32)
        m = c.shape[0]
        lut[n - 2, :m, :] = c
        lut[n - 2, m:2 * m, :] = c[:, ::-1]
    lut = lut.reshape(J - 1, R)
    lut = np.pad(lut, ((0, 0), (0, RP - R)), constant_values=ZROW)
    return np.ascontiguousarray(lut.reshape((J - 1) * (RP // D), D))


_LUT = _build_lut()

_MESH = plsc.VectorSubcoreMesh(core_axis_name="c", subcore_axis_name="s")


@functools.partial(
    pl.kernel,
    out_type=(
        jax.ShapeDtypeStruct((B * R, D), jnp.float32),
        jax.ShapeDtypeStruct((B,), jnp.float32),
    ),
    mesh=_MESH,
    scratch_types=[
        pltpu.VMEM((BPW,), jnp.int32),
        pltpu.VMEM((BPW,), jnp.float32),
        pltpu.VMEM((4 * 16,), jnp.int32),
        pltpu.VMEM((4 * 16, D), jnp.int32),
        [pltpu.VMEM((CH,), jnp.int32) for _ in range(NB)],
        [pltpu.VMEM((CH, D), jnp.float32) for _ in range(NB)],
        pltpu.VMEM((CH,), jnp.int32),
        pltpu.VMEM((CH, D), jnp.float32),
        pltpu.SemaphoreType.DMA,
        [pltpu.SemaphoreType.DMA for _ in range(NB)],
        [pltpu.SemaphoreType.DMA for _ in range(NB)],
    ],
)
def _pairs_sc(table, lut, jn, out, out_pn,
              jn_v, pn_v, tidx_v, lutrows_v, idx_bufs, row_bufs,
              zidx_v, zbuf, lsem, gsems, wsems):
    wid = lax.axis_index("s") * NC + lax.axis_index("c")
    b0 = wid * BPW

    pltpu.sync_copy(jn.at[pl.ds(b0, BPW)], jn_v)

    nn_vecs = []
    for v in range(BPW // 16):
        jv = jn_v[pl.ds(v * 16, 16)]
        pn_v[pl.ds(v * 16, 16)] = (jv * (jv - 1)).astype(jnp.float32)
        nn_vecs.append(jv * (jv - 1))
    pltpu.sync_copy(pn_v, out_pn.at[pl.ds(b0, BPW)])

    zrow = b0 * (J + 1) + ZROW
    for u in range(CH // 16):
        zidx_v[pl.ds(u * 16, 16)] = jnp.zeros((16,), jnp.int32) + zrow
    pltpu.async_copy(table.at[zidx_v], zbuf, lsem).wait()

    nn_s = [None] * BPW
    gdesc = [None] * NJOB
    wdesc = [None] * NJOB

    def fetch_lut_group(g):
        jv = jn_v[pl.ds(g * 16, 16)]
        for i in range(4):
            tidx_v[pl.ds(i * 16, 16)] = 4 * (jv - 2) + i
        pltpu.async_copy(lut.at[tidx_v], lutrows_v, lsem).wait()

    def complete(j):
        ev, c = divmod(j, NCH)
        s = j % NB
        b = b0 + ev
        dst = out.at[pl.ds(b * R + c * CH, CH)]
        if c == 0:
            gdesc[j].wait()
            wdesc[j] = pltpu.async_copy(row_bufs[s], dst, wsems[s])
        else:
            nn = nn_s[ev]

            @pl.when(nn > 48 * c)
            def _():
                gdesc[j].wait()
                pltpu.async_copy(row_bufs[s], dst, wsems[s])

            @pl.when(nn <= 48 * c)
            def _():
                pltpu.async_copy(zbuf, dst, wsems[s])

            wdesc[j] = pltpu.make_async_copy(zbuf, dst, wsems[s])

    for j in range(NJOB):
        ev, c = divmod(j, NCH)
        s = j % NB
        g, k = divmod(ev, 16)
        if c == 0:
            if k == 0:
                fetch_lut_group(g)
            nn_s[ev] = nn_vecs[g][k]
        if j >= NB:
            wdesc[j - NB].wait()

        boff = (b0 + ev) * (J + 1)

        def build_and_gather(c=c, s=s, k=k, boff=boff):
            for u in range(CH // 16):
                v = c * (CH // 16) + u
                i, w = divmod(v, D // 16)
                idx_bufs[s][pl.ds(u * 16, 16)] = (
                    lutrows_v[i * 16 + k, pl.ds(w * 16, 16)] + boff)

        dsc = pltpu.make_async_copy(table.at[idx_bufs[s]], row_bufs[s],
                                    gsems[s])
        if c == 0:
            build_and_gather()
            dsc.start()
        else:
            @pl.when(nn_s[ev] > 48 * c)
            def _(dsc=dsc):
                build_and_gather()
                dsc.start()
        gdesc[j] = dsc

        if j >= PD:
            complete(j - PD)

    for j in range(NJOB - PD, NJOB):
        complete(j)
    for j in range(NJOB - NB, NJOB):
        wdesc[j].wait()


def kernel(inputs, dict_vals, jet_num):
    del dict_vals
    jn32 = jet_num.astype(jnp.int32)
    table = jnp.concatenate(
        [inputs, jnp.zeros((B, 1, D), inputs.dtype)], axis=1
    ).reshape(B * (J + 1), D)
    out_rows, pn = _pairs_sc(table, jnp.asarray(_LUT), jn32)
    return out_rows.reshape(B, 2 * P, 2 * D), pn[:, None]
